# baseline (speedup 1.0000x reference)
import functools

import jax
import jax.numpy as jnp
from jax import lax
from jax.experimental import pallas as pl
from jax.experimental.pallas import tpu as pltpu
from jax.experimental.pallas import tpu_sc as plsc

Q_MIN = -128.0
Q_MAX = 127.0
_MAGIC = 1.5 * (2.0 ** 23)
_PRE = 1024.0


@functools.cache
def _build(V, D, B):
  info = plsc.get_sparse_core_info()
  NC, NS, L = info.num_cores, info.num_subcores, info.num_lanes
  NW = NC * NS
  assert D % L == 0 and B % (8 * NW) == 0
  b_per_w = B // NW
  mesh = plsc.VectorSubcoreMesh(core_axis_name="c", subcore_axis_name="s")

  @functools.partial(
      pl.kernel,
      out_type=jax.ShapeDtypeStruct((B, D), jnp.float32),
      mesh=mesh,
      compiler_params=pltpu.CompilerParams(use_tc_tiling_on_sc=False),
      scratch_types=[
          pltpu.VMEM((b_per_w,), jnp.int32),
          pltpu.VMEM((b_per_w, D), jnp.float32),
          pltpu.VMEM((b_per_w + L,), jnp.float32),
          pltpu.SemaphoreType.DMA,
          pltpu.SemaphoreType.DMA,
      ],
  )
  def dequant_gather(table_hbm, idx_hbm, scales_hbm, out_hbm,
                     idx_v, rows_v, sc_v, sem_rows, sem_sc):
    wid = lax.axis_index("s") * NC + lax.axis_index("c")
    base = wid * b_per_w
    pltpu.sync_copy(idx_hbm.at[pl.ds(base, b_per_w)], idx_v)
    rows_cp = pltpu.async_copy(table_hbm.at[idx_v], rows_v, sem_rows)
    sc_cp = pltpu.async_copy(scales_hbm.at[idx_v], sc_v.at[pl.ds(0, b_per_w)],
                             sem_sc)
    rows_cp.wait()
    sc_cp.wait()

    def row_body(r, carry):
      s = sc_v[pl.ds(r, L)][0]
      for j in range(D // L):
        v = rows_v[r, pl.ds(j * L, L)]
        v = jnp.minimum(jnp.maximum(v, -_PRE), _PRE)
        v = (v + _MAGIC) - _MAGIC
        v = jnp.minimum(jnp.maximum(v, Q_MIN), Q_MAX)
        rows_v[r, pl.ds(j * L, L)] = v * s
      return carry

    lax.fori_loop(0, b_per_w, row_body, 0, unroll=2)
    pltpu.sync_copy(rows_v, out_hbm.at[pl.ds(base, b_per_w)])

  return dequant_gather


def kernel(x, weights, scales):
  V, D = weights.shape
  (B,) = x.shape
  return _build(V, D, B)(weights, x.astype(jnp.int32), scales)
